# baseline (speedup 1.0000x reference)
import functools

import jax
import jax.numpy as jnp
import numpy as np
from jax import lax
from jax.experimental import pallas as pl
from jax.experimental.pallas import tpu as pltpu
from jax.experimental.pallas import tpu_sc as plsc

N_NODES = 10000
N_EDGES = 320000
D = 128
DH = D // 2

NC = 2
NS = 16
NW = NC * NS

CHUNK = 128
EDGES_PER_TILE = 20480
NCHUNKS = EDGES_PER_TILE // CHUNK
EPAD = NS * EDGES_PER_TILE

NPAD = 10240
ROWS_PER_TILE = NPAD // NS
SLABS = ROWS_PER_TILE // CHUNK
XROWS_PER_TILE = N_NODES // NS

NBUF = 4
NFBUF = 2
IH = 80
NPIECES = NCHUNKS // IH

_Q = np.empty((DH,), dtype=np.int32)
for _g in range(DH // 32):
    for _i in range(16):
        _Q[32 * _g + _i] = 32 * _g + 2 * _i
        _Q[32 * _g + 16 + _i] = 32 * _g + 2 * _i + 1


def _sc_scatter_body(src_hbm, dst_hbm, x_hbm, out_hbm,
                     src_v, dst_v, b0_v, b1_v, b2_v, b3_v, f0_v, f1_v,
                     agg_sh, x_sh, g0, g1, g2, g3, t0, t1):
    bufs = [b0_v, b1_v, b2_v, b3_v]
    fbufs = [f0_v, f1_v]
    gsems = [g0, g1, g2, g3]
    ssems = [t0, t1]
    c = lax.axis_index("c")
    s = lax.axis_index("s")

    xr0 = s * XROWS_PER_TILE
    pltpu.sync_copy(x_hbm.at[c].at[pl.ds(xr0, XROWS_PER_TILE)],
                    x_sh.at[pl.ds(xr0, XROWS_PER_TILE)])

    def zbody(i, _):
        f0_v[i // (DH // 16), pl.ds((i % (DH // 16)) * 16, 16)] = (
            jnp.zeros((16,), jnp.float32))
        return 0
    lax.fori_loop(0, CHUNK * (DH // 16), zbody, 0)

    def zslab(k, _):
        pltpu.sync_copy(f0_v,
                        agg_sh.at[pl.ds(s * ROWS_PER_TILE + k * CHUNK, CHUNK)])
        return 0
    lax.fori_loop(0, SLABS, zslab, 0)
    plsc.subcore_barrier()

    def g_start(j, b):
        pltpu.async_copy(x_sh.at[src_v.at[j]], bufs[b], gsems[b])

    def g_wait(j, b):
        pltpu.make_async_copy(x_sh.at[src_v.at[j]], bufs[b],
                              gsems[b]).wait()

    def s_start(j, f):
        pltpu.async_copy(fbufs[f], agg_sh.at[dst_v.at[j]], ssems[f], add=True)

    def s_wait(j, f):
        pltpu.make_async_copy(fbufs[f], agg_sh.at[dst_v.at[j]],
                              ssems[f]).wait()

    def widen(b, f):
        def wbody(r, _):
            for g in range(DH // 32):
                words = bufs[b][r, pl.ds(16 * g, 16)]
                lo = lax.bitcast_convert_type(words << 16, jnp.float32)
                hi = lax.bitcast_convert_type(words & jnp.int32(-65536),
                                              jnp.float32)
                fbufs[f][r, pl.ds(32 * g, 16)] = lo
                fbufs[f][r, pl.ds(32 * g + 16, 16)] = hi
            return 0
        lax.fori_loop(0, CHUNK, wbody, 0)

    def piece(p, _):
        pltpu.sync_copy(src_hbm.at[s].at[pl.ds(p * IH, IH)], src_v)
        pltpu.sync_copy(dst_hbm.at[s].at[pl.ds(p * IH, IH)], dst_v)

        g_start(0, 0)
        g_start(1, 1)

        def ebody(i, _):
            j0 = i * NBUF
            for b in range(NBUF):
                j = j0 + b
                b2 = (b + 2) % NBUF
                f = b % NFBUF
                g_wait(j, b)

                @pl.when(j >= 2)
                def _():
                    s_wait(j - 2, f)

                widen(b, f)
                s_start(j, f)

                @pl.when(j + 2 < IH)
                def _():
                    g_start(j + 2, b2)
            return 0
        lax.fori_loop(0, IH // NBUF, ebody, 0)
        s_wait(IH - 2, (IH - 2) % NFBUF)
        s_wait(IH - 1, (IH - 1) % NFBUF)
        return 0
    lax.fori_loop(0, NPIECES, piece, 0)
    plsc.subcore_barrier()

    row0 = s * ROWS_PER_TILE
    pltpu.sync_copy(agg_sh.at[pl.ds(row0, ROWS_PER_TILE)],
                    out_hbm.at[c].at[pl.ds(row0, ROWS_PER_TILE)])


@functools.cache
def _sc_scatter_kernel():
    return pl.kernel(
        _sc_scatter_body,
        out_type=jax.ShapeDtypeStruct((NC, NPAD, DH), jnp.float32),
        mesh=plsc.VectorSubcoreMesh(core_axis_name="c", subcore_axis_name="s",
                                    num_cores=NC, num_subcores=NS),
        scratch_types=[
            pltpu.VMEM((IH, CHUNK), jnp.int32),
            pltpu.VMEM((IH, CHUNK), jnp.int32),
            pltpu.VMEM((CHUNK, DH // 2), jnp.int32),
            pltpu.VMEM((CHUNK, DH // 2), jnp.int32),
            pltpu.VMEM((CHUNK, DH // 2), jnp.int32),
            pltpu.VMEM((CHUNK, DH // 2), jnp.int32),
            pltpu.VMEM((CHUNK, DH), jnp.float32),
            pltpu.VMEM((CHUNK, DH), jnp.float32),
            pltpu.VMEM_SHARED((NPAD, DH), jnp.float32),
            pltpu.VMEM_SHARED((N_NODES, DH // 2), jnp.int32),
        ] + [pltpu.SemaphoreType.DMA] * 6,
        compiler_params=pltpu.CompilerParams(use_tc_tiling_on_sc=False),
    )


def _sc_scatter(src3, dst3, x_split):
    return _sc_scatter_kernel()(src3, dst3, x_split)


def _tc_layer_body(agg_ref, x_ref, wrel_a_ref, wrel_b_ref, wroot_ref, b_ref,
                   o_ref, osplit_ref):
    acc = jnp.dot(agg_ref[0], wrel_a_ref[...],
                  preferred_element_type=jnp.float32)
    acc = acc + jnp.dot(agg_ref[1], wrel_b_ref[...],
                        preferred_element_type=jnp.float32)
    acc = acc + jnp.dot(x_ref[...], wroot_ref[...],
                        preferred_element_type=jnp.float32)
    acc = jnp.maximum(acc + b_ref[...], 0.0)
    o_ref[...] = acc
    if osplit_ref is not None:
        osplit_ref[0] = acc[:, :DH].astype(jnp.bfloat16)
        osplit_ref[1] = acc[:, DH:].astype(jnp.bfloat16)


def _tc_layer(agg, x, wrel_t, wroot_t, b, want_split):
    nb, bl = 5, N_NODES // 5
    out_shape = [jax.ShapeDtypeStruct((N_NODES, D), jnp.float32)]
    out_specs = [pl.BlockSpec((bl, D), lambda i: (i, 0))]
    if want_split:
        out_shape.append(jax.ShapeDtypeStruct((NC, N_NODES, DH), jnp.bfloat16))
        out_specs.append(pl.BlockSpec((NC, bl, DH), lambda i: (0, i, 0)))
        body = _tc_layer_body
    else:
        body = functools.partial(_tc_layer_body, osplit_ref=None)
    return pl.pallas_call(
        body,
        grid=(nb,),
        in_specs=[
            pl.BlockSpec((NC, bl, DH), lambda i: (0, i, 0)),
            pl.BlockSpec((bl, D), lambda i: (i, 0)),
            pl.BlockSpec((DH, D), lambda i: (0, 0)),
            pl.BlockSpec((DH, D), lambda i: (0, 0)),
            pl.BlockSpec((D, D), lambda i: (0, 0)),
            pl.BlockSpec((1, D), lambda i: (0, 0)),
        ],
        out_specs=out_specs,
        out_shape=out_shape,
    )(agg, x, wrel_t[:DH][_Q], wrel_t[DH:][_Q], wroot_t, b)


def _pack_halves(a_bf):
    return lax.bitcast_convert_type(
        a_bf.reshape(NC, N_NODES, DH // 2, 2), jnp.int32)


def kernel(x, edge_index, W1_rel, b1, W1_root, W2_rel, b2, W2_root):
    ei = edge_index.astype(jnp.int32)
    pad = EPAD - N_EDGES
    src3 = jnp.concatenate(
        [ei[0], jnp.zeros((pad,), jnp.int32)]).reshape(NS, NCHUNKS, CHUNK)
    dst3 = jnp.concatenate(
        [ei[1], jnp.full((pad,), NPAD - 1, jnp.int32)]).reshape(NS, NCHUNKS, CHUNK)

    x_split = _pack_halves(
        x.astype(jnp.bfloat16).reshape(N_NODES, NC, DH).transpose(1, 0, 2))
    agg1 = _sc_scatter(src3, dst3, x_split)
    h, h_split = _tc_layer(agg1, x, W1_rel.T, W1_root.T, b1.reshape(1, -1),
                           want_split=True)
    agg2 = _sc_scatter(src3, dst3, _pack_halves(h_split))
    (out,) = _tc_layer(agg2, h, W2_rel.T, W2_root.T, b2.reshape(1, -1),
                       want_split=False)
    return out


# speedup vs baseline: 1.5348x; 1.5348x over previous
import functools

import jax
import jax.numpy as jnp
from jax import lax
from jax.experimental import pallas as pl
from jax.experimental.pallas import tpu as pltpu
from jax.experimental.pallas import tpu_sc as plsc

N_NODES = 10000
N_EDGES = 320000
D = 128
DH = D // 2

NC = 2
NS = 16
NW = NC * NS

CHUNK = 128
EDGES_PER_TILE = 20480
NCHUNKS = EDGES_PER_TILE // CHUNK
EPAD = NS * EDGES_PER_TILE

NPAD = 10240
ROWS_PER_TILE = NPAD // NS
SLABS = ROWS_PER_TILE // CHUNK
XROWS_PER_TILE = N_NODES // NS

NBUF = 4
IH = 20
NPIECES = NCHUNKS // IH


def _sc_scatter_body(src_hbm, dst_hbm, x_hbm, out_hbm,
                     sv0, sv1, dv0, dv1, b0_v, b1_v, b2_v, b3_v, agg_sh, x_sh,
                     g0, g1, g2, g3, t0, t1, t2, t3, i0, i1, xsem):
    bufs = [b0_v, b1_v, b2_v, b3_v]
    src_bufs = [sv0, sv1]
    dst_bufs = [dv0, dv1]
    gsems = [g0, g1, g2, g3]
    ssems = [t0, t1, t2, t3]
    isems = [i0, i1]
    c = lax.axis_index("c")
    s = lax.axis_index("s")

    xr0 = s * XROWS_PER_TILE
    xstage = pltpu.make_async_copy(x_hbm.at[c].at[pl.ds(xr0, XROWS_PER_TILE)],
                                   x_sh.at[pl.ds(xr0, XROWS_PER_TILE)], xsem)
    xstage.start()

    def i_start(p, ib):
        pltpu.async_copy(src_hbm.at[s].at[pl.ds(p * IH, IH)], src_bufs[ib],
                         isems[ib])
        pltpu.async_copy(dst_hbm.at[s].at[pl.ds(p * IH, IH)], dst_bufs[ib],
                         isems[ib])

    def i_wait(p, ib):
        pltpu.make_async_copy(src_hbm.at[s].at[pl.ds(p * IH, IH)],
                              src_bufs[ib], isems[ib]).wait()
        pltpu.make_async_copy(dst_hbm.at[s].at[pl.ds(p * IH, IH)],
                              dst_bufs[ib], isems[ib]).wait()

    i_start(0, 0)

    def zbody(i, _):
        b0_v[i // (DH // 16), pl.ds((i % (DH // 16)) * 16, 16)] = (
            jnp.zeros((16,), jnp.float32))
        return 0
    lax.fori_loop(0, CHUNK * (DH // 16), zbody, 0)

    def zslab(k, _):
        pltpu.sync_copy(b0_v,
                        agg_sh.at[pl.ds(s * ROWS_PER_TILE + k * CHUNK, CHUNK)])
        return 0
    lax.fori_loop(0, SLABS, zslab, 0)
    xstage.wait()
    plsc.subcore_barrier()

    def g_start(j, b, src_v):
        pltpu.async_copy(x_sh.at[src_v.at[j]], bufs[b], gsems[b])

    def g_wait(j, b, src_v):
        pltpu.make_async_copy(x_sh.at[src_v.at[j]], bufs[b],
                              gsems[b]).wait()

    def s_start(j, b, dst_v):
        pltpu.async_copy(bufs[b], agg_sh.at[dst_v.at[j]], ssems[b], add=True)

    def s_wait(j, b, dst_v):
        pltpu.make_async_copy(bufs[b], agg_sh.at[dst_v.at[j]],
                              ssems[b]).wait()

    for p in range(NPIECES):
        ib = p % 2
        src_v = src_bufs[ib]
        dst_v = dst_bufs[ib]
        i_wait(p, ib)
        if p + 1 < NPIECES:
            i_start(p + 1, 1 - ib)

        g_start(0, 0, src_v)
        g_start(1, 1, src_v)

        def ebody(i, _, src_v=src_v, dst_v=dst_v):
            j0 = i * NBUF
            for b in range(NBUF):
                j = j0 + b
                b2 = (b + 2) % NBUF
                g_wait(j, b, src_v)
                s_start(j, b, dst_v)

                @pl.when(j >= 2)
                def _():
                    s_wait(j - 2, b2, dst_v)

                @pl.when(j + 2 < IH)
                def _():
                    g_start(j + 2, b2, src_v)
            return 0
        lax.fori_loop(0, IH // NBUF, ebody, 0)
        s_wait(IH - 2, (IH - 2) % NBUF, dst_v)
        s_wait(IH - 1, (IH - 1) % NBUF, dst_v)
    plsc.subcore_barrier()

    row0 = s * ROWS_PER_TILE
    pltpu.sync_copy(agg_sh.at[pl.ds(row0, ROWS_PER_TILE)],
                    out_hbm.at[c].at[pl.ds(row0, ROWS_PER_TILE)])


@functools.cache
def _sc_scatter_kernel():
    return pl.kernel(
        _sc_scatter_body,
        out_type=jax.ShapeDtypeStruct((NC, NPAD, DH), jnp.float32),
        mesh=plsc.VectorSubcoreMesh(core_axis_name="c", subcore_axis_name="s",
                                    num_cores=NC, num_subcores=NS),
        scratch_types=[
            pltpu.VMEM((IH, CHUNK), jnp.int32),
            pltpu.VMEM((IH, CHUNK), jnp.int32),
            pltpu.VMEM((IH, CHUNK), jnp.int32),
            pltpu.VMEM((IH, CHUNK), jnp.int32),
            pltpu.VMEM((CHUNK, DH), jnp.float32),
            pltpu.VMEM((CHUNK, DH), jnp.float32),
            pltpu.VMEM((CHUNK, DH), jnp.float32),
            pltpu.VMEM((CHUNK, DH), jnp.float32),
            pltpu.VMEM_SHARED((NPAD, DH), jnp.float32),
            pltpu.VMEM_SHARED((N_NODES, DH), jnp.float32),
        ] + [pltpu.SemaphoreType.DMA] * 11,
        compiler_params=pltpu.CompilerParams(use_tc_tiling_on_sc=False),
    )


def _sc_scatter(src3, dst3, x_split):
    return _sc_scatter_kernel()(src3, dst3, x_split)


def _tc_layer_body(agg_ref, x_ref, wrel_a_ref, wrel_b_ref, wroot_ref, b_ref,
                   o_ref, osplit_ref):
    acc = jnp.dot(agg_ref[0], wrel_a_ref[...],
                  preferred_element_type=jnp.float32)
    acc = acc + jnp.dot(agg_ref[1], wrel_b_ref[...],
                        preferred_element_type=jnp.float32)
    acc = acc + jnp.dot(x_ref[...], wroot_ref[...],
                        preferred_element_type=jnp.float32)
    acc = jnp.maximum(acc + b_ref[...], 0.0)
    o_ref[...] = acc
    if osplit_ref is not None:
        osplit_ref[0] = acc[:, :DH]
        osplit_ref[1] = acc[:, DH:]


def _tc_layer(agg, x, wrel_t, wroot_t, b, want_split):
    nb, bl = 5, N_NODES // 5
    out_shape = [jax.ShapeDtypeStruct((N_NODES, D), jnp.float32)]
    out_specs = [pl.BlockSpec((bl, D), lambda i: (i, 0))]
    if want_split:
        out_shape.append(jax.ShapeDtypeStruct((NC, N_NODES, DH), jnp.float32))
        out_specs.append(pl.BlockSpec((NC, bl, DH), lambda i: (0, i, 0)))
        body = _tc_layer_body
    else:
        body = functools.partial(_tc_layer_body, osplit_ref=None)
    return pl.pallas_call(
        body,
        grid=(nb,),
        in_specs=[
            pl.BlockSpec((NC, bl, DH), lambda i: (0, i, 0)),
            pl.BlockSpec((bl, D), lambda i: (i, 0)),
            pl.BlockSpec((DH, D), lambda i: (0, 0)),
            pl.BlockSpec((DH, D), lambda i: (0, 0)),
            pl.BlockSpec((D, D), lambda i: (0, 0)),
            pl.BlockSpec((1, D), lambda i: (0, 0)),
        ],
        out_specs=out_specs,
        out_shape=out_shape,
    )(agg, x, wrel_t[:DH], wrel_t[DH:], wroot_t, b)


def kernel(x, edge_index, W1_rel, b1, W1_root, W2_rel, b2, W2_root):
    ei = edge_index.astype(jnp.int32)
    pad = EPAD - N_EDGES
    src3 = jnp.concatenate(
        [ei[0], jnp.zeros((pad,), jnp.int32)]).reshape(NS, NCHUNKS, CHUNK)
    dst3 = jnp.concatenate(
        [ei[1], jnp.full((pad,), NPAD - 1, jnp.int32)]).reshape(NS, NCHUNKS, CHUNK)

    x_split = x.reshape(N_NODES, NC, DH).transpose(1, 0, 2)
    agg1 = _sc_scatter(src3, dst3, x_split)
    h, h_split = _tc_layer(agg1, x, W1_rel.T, W1_root.T, b1.reshape(1, -1),
                           want_split=True)
    agg2 = _sc_scatter(src3, dst3, h_split)
    (out,) = _tc_layer(agg2, h, W2_rel.T, W2_root.T, b2.reshape(1, -1),
                       want_split=False)
    return out


# speedup vs baseline: 1.5452x; 1.0068x over previous
import functools

import jax
import jax.numpy as jnp
from jax import lax
from jax.experimental import pallas as pl
from jax.experimental.pallas import tpu as pltpu
from jax.experimental.pallas import tpu_sc as plsc

N_NODES = 10000
N_EDGES = 320000
D = 128
DH = D // 2

NC = 2
NS = 16
NW = NC * NS

CHUNK = 128
EDGES_PER_TILE = 20480
NCHUNKS = EDGES_PER_TILE // CHUNK
EPAD = NS * EDGES_PER_TILE

NPAD = 10240
ROWS_PER_TILE = NPAD // NS
SLABS = ROWS_PER_TILE // CHUNK
XROWS_PER_TILE = N_NODES // NS

NBUF = 4
IH = 40
NPIECES = NCHUNKS // IH


def _sc_scatter_body(src_hbm, dst_hbm, x_hbm, out_hbm,
                     src_v, dst_v, b0_v, b1_v, b2_v, b3_v, agg_sh, x_sh,
                     g0, g1, g2, g3, t0, t1, t2, t3):
    bufs = [b0_v, b1_v, b2_v, b3_v]
    gsems = [g0, g1, g2, g3]
    ssems = [t0, t1, t2, t3]
    c = lax.axis_index("c")
    s = lax.axis_index("s")

    xr0 = s * XROWS_PER_TILE
    pltpu.sync_copy(x_hbm.at[c].at[pl.ds(xr0, XROWS_PER_TILE)],
                    x_sh.at[pl.ds(xr0, XROWS_PER_TILE)])

    def zbody(i, _):
        b0_v[i // (DH // 16), pl.ds((i % (DH // 16)) * 16, 16)] = (
            jnp.zeros((16,), jnp.float32))
        return 0
    lax.fori_loop(0, CHUNK * (DH // 16), zbody, 0)

    def zslab(k, _):
        pltpu.sync_copy(b0_v,
                        agg_sh.at[pl.ds(s * ROWS_PER_TILE + k * CHUNK, CHUNK)])
        return 0
    lax.fori_loop(0, SLABS, zslab, 0)
    plsc.subcore_barrier()

    def g_start(j, b):
        pltpu.async_copy(x_sh.at[src_v.at[j]], bufs[b], gsems[b])

    def g_wait(j, b):
        pltpu.make_async_copy(x_sh.at[src_v.at[j]], bufs[b],
                              gsems[b]).wait()

    def s_start(j, b):
        pltpu.async_copy(bufs[b], agg_sh.at[dst_v.at[j]], ssems[b], add=True)

    def s_wait(j, b):
        pltpu.make_async_copy(bufs[b], agg_sh.at[dst_v.at[j]],
                              ssems[b]).wait()

    def piece(p, _):
        pltpu.sync_copy(src_hbm.at[s].at[pl.ds(p * IH, IH)], src_v)
        pltpu.sync_copy(dst_hbm.at[s].at[pl.ds(p * IH, IH)], dst_v)

        g_start(0, 0)
        g_start(1, 1)

        def ebody(i, _):
            j0 = i * NBUF
            for b in range(NBUF):
                j = j0 + b
                b2 = (b + 2) % NBUF
                g_wait(j, b)
                s_start(j, b)

                @pl.when(j >= 2)
                def _():
                    s_wait(j - 2, b2)

                @pl.when(j + 2 < IH)
                def _():
                    g_start(j + 2, b2)
            return 0
        lax.fori_loop(0, IH // NBUF, ebody, 0)
        s_wait(IH - 2, (IH - 2) % NBUF)
        s_wait(IH - 1, (IH - 1) % NBUF)
        return 0
    lax.fori_loop(0, NPIECES, piece, 0)
    plsc.subcore_barrier()

    row0 = s * ROWS_PER_TILE
    pltpu.sync_copy(agg_sh.at[pl.ds(row0, ROWS_PER_TILE)],
                    out_hbm.at[c].at[pl.ds(row0, ROWS_PER_TILE)])


@functools.cache
def _sc_scatter_kernel():
    return pl.kernel(
        _sc_scatter_body,
        out_type=jax.ShapeDtypeStruct((NC, NPAD, DH), jnp.float32),
        mesh=plsc.VectorSubcoreMesh(core_axis_name="c", subcore_axis_name="s",
                                    num_cores=NC, num_subcores=NS),
        scratch_types=[
            pltpu.VMEM((IH, CHUNK), jnp.int32),
            pltpu.VMEM((IH, CHUNK), jnp.int32),
            pltpu.VMEM((CHUNK, DH), jnp.float32),
            pltpu.VMEM((CHUNK, DH), jnp.float32),
            pltpu.VMEM((CHUNK, DH), jnp.float32),
            pltpu.VMEM((CHUNK, DH), jnp.float32),
            pltpu.VMEM_SHARED((NPAD, DH), jnp.float32),
            pltpu.VMEM_SHARED((N_NODES, DH), jnp.float32),
        ] + [pltpu.SemaphoreType.DMA] * 8,
        compiler_params=pltpu.CompilerParams(use_tc_tiling_on_sc=False),
    )


def _sc_scatter(src3, dst3, x_split):
    return _sc_scatter_kernel()(src3, dst3, x_split)


def _tc_layer_body(agg_ref, x_ref, wrel_a_ref, wrel_b_ref, wroot_ref, b_ref,
                   o_ref, osplit_ref):
    acc = jnp.dot(agg_ref[0], wrel_a_ref[...],
                  preferred_element_type=jnp.float32)
    acc = acc + jnp.dot(agg_ref[1], wrel_b_ref[...],
                        preferred_element_type=jnp.float32)
    acc = acc + jnp.dot(x_ref[...], wroot_ref[...],
                        preferred_element_type=jnp.float32)
    acc = jnp.maximum(acc + b_ref[...], 0.0)
    o_ref[...] = acc
    if osplit_ref is not None:
        osplit_ref[0] = acc[:, :DH]
        osplit_ref[1] = acc[:, DH:]


def _tc_layer(agg, x, wrel_t, wroot_t, b, want_split):
    nb, bl = 5, N_NODES // 5
    out_shape = [jax.ShapeDtypeStruct((N_NODES, D), jnp.float32)]
    out_specs = [pl.BlockSpec((bl, D), lambda i: (i, 0))]
    if want_split:
        out_shape.append(jax.ShapeDtypeStruct((NC, N_NODES, DH), jnp.float32))
        out_specs.append(pl.BlockSpec((NC, bl, DH), lambda i: (0, i, 0)))
        body = _tc_layer_body
    else:
        body = functools.partial(_tc_layer_body, osplit_ref=None)
    return pl.pallas_call(
        body,
        grid=(nb,),
        in_specs=[
            pl.BlockSpec((NC, bl, DH), lambda i: (0, i, 0)),
            pl.BlockSpec((bl, D), lambda i: (i, 0)),
            pl.BlockSpec((DH, D), lambda i: (0, 0)),
            pl.BlockSpec((DH, D), lambda i: (0, 0)),
            pl.BlockSpec((D, D), lambda i: (0, 0)),
            pl.BlockSpec((1, D), lambda i: (0, 0)),
        ],
        out_specs=out_specs,
        out_shape=out_shape,
    )(agg, x, wrel_t[:DH], wrel_t[DH:], wroot_t, b)


def kernel(x, edge_index, W1_rel, b1, W1_root, W2_rel, b2, W2_root):
    ei = edge_index.astype(jnp.int32)
    pad = EPAD - N_EDGES
    src3 = jnp.concatenate(
        [ei[0], jnp.zeros((pad,), jnp.int32)]).reshape(NS, NCHUNKS, CHUNK)
    dst3 = jnp.concatenate(
        [ei[1], jnp.full((pad,), NPAD - 1, jnp.int32)]).reshape(NS, NCHUNKS, CHUNK)

    x_split = x.reshape(N_NODES, NC, DH).transpose(1, 0, 2)
    agg1 = _sc_scatter(src3, dst3, x_split)
    h, h_split = _tc_layer(agg1, x, W1_rel.T, W1_root.T, b1.reshape(1, -1),
                           want_split=True)
    agg2 = _sc_scatter(src3, dst3, h_split)
    (out,) = _tc_layer(agg2, h, W2_rel.T, W2_root.T, b2.reshape(1, -1),
                       want_split=False)
    return out
